# Initial kernel scaffold; baseline (speedup 1.0000x reference)
import jax
import jax.numpy as jnp
from jax.experimental import pallas as pl


def kernel(input_ids, embedding_weight):
    raise NotImplementedError("write your pallas kernel here")


# speedup vs baseline: 5.1659x; 5.1659x over previous
import functools

import jax
import jax.numpy as jnp
from jax import lax
from jax.experimental import pallas as pl
from jax.experimental.pallas import tpu as pltpu
from jax.experimental.pallas import tpu_sc as plsc

HIDDEN = 128
NUM_CORES = 2
NUM_SUBCORES = 16
NUM_WORKERS = NUM_CORES * NUM_SUBCORES

CHUNK = 128


def _make_gather(total_rows: int):
    assert total_rows % (NUM_WORKERS * CHUNK) == 0
    rows_per_worker = total_rows // NUM_WORKERS
    n_steps = rows_per_worker // CHUNK

    mesh = plsc.VectorSubcoreMesh(core_axis_name="c", subcore_axis_name="s")

    @functools.partial(
        pl.kernel,
        mesh=mesh,
        out_type=jax.ShapeDtypeStruct((total_rows, HIDDEN), jnp.float32),
        scratch_types=[
            pltpu.VMEM((CHUNK,), jnp.int32),
            pltpu.VMEM((CHUNK, HIDDEN), jnp.float32),
            pltpu.SemaphoreType.DMA,
        ],
    )
    def gather_kernel(table_hbm, idx_hbm, out_hbm, idx_v, rows_v, sem):
        wid = lax.axis_index("s") * NUM_CORES + lax.axis_index("c")
        base0 = wid * rows_per_worker

        def body(i, carry):
            base = base0 + i * CHUNK
            pltpu.sync_copy(idx_hbm.at[pl.ds(base, CHUNK)], idx_v)
            pltpu.async_copy(table_hbm.at[idx_v], rows_v, sem).wait()
            pltpu.sync_copy(rows_v, out_hbm.at[pl.ds(base, CHUNK)])
            return carry

        lax.fori_loop(0, n_steps, body, 0)

    return gather_kernel


def kernel(input_ids, embedding_weight):
    batch, seq = input_ids.shape
    ids = input_ids.reshape(-1).astype(jnp.int32)
    out = _make_gather(ids.shape[0])(embedding_weight, ids)
    return out.reshape(batch, seq, HIDDEN)


# speedup vs baseline: 8.2907x; 1.6049x over previous
import functools

import jax
import jax.numpy as jnp
from jax import lax
from jax.experimental import pallas as pl
from jax.experimental.pallas import tpu as pltpu
from jax.experimental.pallas import tpu_sc as plsc

HIDDEN = 128
NUM_CORES = 2
NUM_SUBCORES = 16
NUM_WORKERS = NUM_CORES * NUM_SUBCORES

CHUNK = 128
K = 2
SLAB = K * CHUNK


def _make_gather(total_rows: int):
    assert total_rows % (NUM_WORKERS * SLAB) == 0
    rows_per_worker = total_rows // NUM_WORKERS
    n_slabs = rows_per_worker // SLAB
    assert n_slabs % 2 == 0
    half = n_slabs // 2

    mesh = plsc.VectorSubcoreMesh(core_axis_name="c", subcore_axis_name="s")

    @functools.partial(
        pl.kernel,
        mesh=mesh,
        out_type=jax.ShapeDtypeStruct((total_rows, HIDDEN), jnp.float32),
        scratch_types=[
            pltpu.VMEM((SLAB,), jnp.int32),
            pltpu.VMEM((SLAB,), jnp.int32),
            pltpu.VMEM((SLAB, HIDDEN), jnp.float32),
            pltpu.VMEM((SLAB, HIDDEN), jnp.float32),
            pltpu.SemaphoreType.DMA,
            pltpu.SemaphoreType.DMA,
            pltpu.SemaphoreType.DMA,
            pltpu.SemaphoreType.DMA,
        ],
    )
    def gather_kernel(table_hbm, idx_hbm, out_hbm, idx0, idx1, rows0, rows1,
                      g0, g1, w0, w1):
        wid = lax.axis_index("s") * NUM_CORES + lax.axis_index("c")
        base0 = wid * rows_per_worker

        def issue(slab_i, idx_v, rows_v, gsem):
            row = base0 + slab_i * SLAB
            pltpu.sync_copy(idx_hbm.at[pl.ds(row, SLAB)], idx_v)
            for j in range(K):
                pltpu.async_copy(
                    table_hbm.at[idx_v.at[pl.ds(j * CHUNK, CHUNK)]],
                    rows_v.at[pl.ds(j * CHUNK, CHUNK)],
                    gsem,
                )

        def drain(rows_v, gsem):
            for j in range(K):
                pltpu.make_async_copy(
                    table_hbm.at[pl.ds(0, CHUNK)],
                    rows_v.at[pl.ds(j * CHUNK, CHUNK)],
                    gsem,
                ).wait()

        def writeback(slab_i, rows_v, wsem):
            row = base0 + slab_i * SLAB
            pltpu.async_copy(rows_v, out_hbm.at[pl.ds(row, SLAB)], wsem)

        def wait_writeback(rows_v, wsem):
            pltpu.make_async_copy(
                rows_v, out_hbm.at[pl.ds(0, SLAB)], wsem
            ).wait()

        issue(0, idx0, rows0, g0)

        def body(t, carry):
            a = 2 * t
            drain(rows0, g0)
            writeback(a, rows0, w0)
            @pl.when(t > 0)
            def _():
                wait_writeback(rows1, w1)
            issue(a + 1, idx1, rows1, g1)
            drain(rows1, g1)
            writeback(a + 1, rows1, w1)
            wait_writeback(rows0, w0)

            @pl.when(t < half - 1)
            def _():
                issue(a + 2, idx0, rows0, g0)

            return carry

        lax.fori_loop(0, half, body, 0)
        wait_writeback(rows1, w1)

    return gather_kernel


def kernel(input_ids, embedding_weight):
    batch, seq = input_ids.shape
    ids = input_ids.reshape(-1).astype(jnp.int32)
    out = _make_gather(batch * seq)(embedding_weight, ids)
    return out.reshape(batch, seq, HIDDEN)


# speedup vs baseline: 9.1340x; 1.1017x over previous
import functools

import jax
import jax.numpy as jnp
from jax import lax
from jax.experimental import pallas as pl
from jax.experimental.pallas import tpu as pltpu
from jax.experimental.pallas import tpu_sc as plsc

HIDDEN = 128
NUM_CORES = 2
NUM_SUBCORES = 16
NUM_WORKERS = NUM_CORES * NUM_SUBCORES

CHUNK = 128
NBUF = 4
LOOKAHEAD = 2


def _make_gather(total_rows: int):
    assert total_rows % (NUM_WORKERS * CHUNK * NBUF) == 0
    rows_per_worker = total_rows // NUM_WORKERS
    n_chunks = rows_per_worker // CHUNK
    n_steps = n_chunks // NBUF

    mesh = plsc.VectorSubcoreMesh(core_axis_name="c", subcore_axis_name="s")

    @functools.partial(
        pl.kernel,
        mesh=mesh,
        out_type=jax.ShapeDtypeStruct((total_rows, HIDDEN), jnp.float32),
        scratch_types=[
            pltpu.VMEM((rows_per_worker,), jnp.int32),
        ]
        + [pltpu.VMEM((CHUNK, HIDDEN), jnp.float32) for _ in range(NBUF)]
        + [pltpu.SemaphoreType.DMA for _ in range(2 * NBUF)],
    )
    def gather_kernel(table_hbm, idx_hbm, out_hbm, idx_all, *bufs_and_sems):
        rows = bufs_and_sems[:NBUF]
        gsem = bufs_and_sems[NBUF:2 * NBUF]
        wsem = bufs_and_sems[2 * NBUF:]
        wid = lax.axis_index("s") * NUM_CORES + lax.axis_index("c")
        base0 = wid * rows_per_worker

        pltpu.sync_copy(idx_hbm.at[pl.ds(base0, rows_per_worker)], idx_all)

        def wait_writeback(b):
            pltpu.make_async_copy(
                rows[b], out_hbm.at[pl.ds(0, CHUNK)], wsem[b]
            ).wait()

        def issue_gather(s, b):
            @pl.when(s >= NBUF)
            def _():
                wait_writeback(b)

            pltpu.async_copy(
                table_hbm.at[idx_all.at[pl.ds(s * CHUNK, CHUNK)]],
                rows[b],
                gsem[b],
            )

        def drain_and_writeback(s, b):
            pltpu.make_async_copy(
                table_hbm.at[pl.ds(0, CHUNK)], rows[b], gsem[b]
            ).wait()
            pltpu.async_copy(
                rows[b], out_hbm.at[pl.ds(base0 + s * CHUNK, CHUNK)], wsem[b]
            )

        for s in range(LOOKAHEAD):
            issue_gather(s, s)

        def body(t, carry):
            for b in range(NBUF):
                s = NBUF * t + b
                drain_and_writeback(s, b)

                @pl.when(s + LOOKAHEAD < n_chunks)
                def _():
                    issue_gather(s + LOOKAHEAD, (b + LOOKAHEAD) % NBUF)

            return carry

        lax.fori_loop(0, n_steps, body, 0)
        for b in range(NBUF):
            wait_writeback(b)

    return gather_kernel


def kernel(input_ids, embedding_weight):
    batch, seq = input_ids.shape
    ids = input_ids.reshape(-1).astype(jnp.int32)
    out = _make_gather(batch * seq)(embedding_weight, ids)
    return out.reshape(batch, seq, HIDDEN)


# speedup vs baseline: 9.1381x; 1.0004x over previous
import functools

import jax
import jax.numpy as jnp
from jax import lax
from jax.experimental import pallas as pl
from jax.experimental.pallas import tpu as pltpu
from jax.experimental.pallas import tpu_sc as plsc

HIDDEN = 128
NUM_CORES = 2
NUM_SUBCORES = 16
NUM_WORKERS = NUM_CORES * NUM_SUBCORES

CHUNK = 128
NBUF = 5
LOOKAHEAD = 3


def _make_gather(total_rows: int):
    assert total_rows % (NUM_WORKERS * CHUNK * NBUF) == 0
    rows_per_worker = total_rows // NUM_WORKERS
    n_chunks = rows_per_worker // CHUNK
    n_steps = n_chunks // NBUF

    mesh = plsc.VectorSubcoreMesh(core_axis_name="c", subcore_axis_name="s")

    @functools.partial(
        pl.kernel,
        mesh=mesh,
        out_type=jax.ShapeDtypeStruct((total_rows, HIDDEN), jnp.float32),
        scratch_types=[
            pltpu.VMEM((rows_per_worker,), jnp.int32),
        ]
        + [pltpu.VMEM((CHUNK, HIDDEN), jnp.float32) for _ in range(NBUF)]
        + [pltpu.SemaphoreType.DMA for _ in range(2 * NBUF)],
    )
    def gather_kernel(table_hbm, idx_hbm, out_hbm, idx_all, *bufs_and_sems):
        rows = bufs_and_sems[:NBUF]
        gsem = bufs_and_sems[NBUF:2 * NBUF]
        wsem = bufs_and_sems[2 * NBUF:]
        wid = lax.axis_index("s") * NUM_CORES + lax.axis_index("c")
        base0 = wid * rows_per_worker

        pltpu.sync_copy(idx_hbm.at[pl.ds(base0, rows_per_worker)], idx_all)

        def wait_writeback(b):
            pltpu.make_async_copy(
                rows[b], out_hbm.at[pl.ds(0, CHUNK)], wsem[b]
            ).wait()

        def issue_gather(s, b):
            @pl.when(s >= NBUF)
            def _():
                wait_writeback(b)

            pltpu.async_copy(
                table_hbm.at[idx_all.at[pl.ds(s * CHUNK, CHUNK)]],
                rows[b],
                gsem[b],
            )

        def drain_and_writeback(s, b):
            pltpu.make_async_copy(
                table_hbm.at[pl.ds(0, CHUNK)], rows[b], gsem[b]
            ).wait()
            pltpu.async_copy(
                rows[b], out_hbm.at[pl.ds(base0 + s * CHUNK, CHUNK)], wsem[b]
            )

        for s in range(LOOKAHEAD):
            issue_gather(s, s)

        def body(t, carry):
            for b in range(NBUF):
                s = NBUF * t + b
                drain_and_writeback(s, b)

                @pl.when(s + LOOKAHEAD < n_chunks)
                def _():
                    issue_gather(s + LOOKAHEAD, (b + LOOKAHEAD) % NBUF)

            return carry

        lax.fori_loop(0, n_steps, body, 0)
        for b in range(NBUF):
            wait_writeback(b)

    return gather_kernel


def kernel(input_ids, embedding_weight):
    batch, seq = input_ids.shape
    ids = input_ids.reshape(-1).astype(jnp.int32)
    out = _make_gather(batch * seq)(embedding_weight, ids)
    return out.reshape(batch, seq, HIDDEN)


# speedup vs baseline: 9.1449x; 1.0007x over previous
import functools

import jax
import jax.numpy as jnp
from jax import lax
from jax.experimental import pallas as pl
from jax.experimental.pallas import tpu as pltpu
from jax.experimental.pallas import tpu_sc as plsc

HIDDEN = 128
NUM_CORES = 2
NUM_SUBCORES = 16
NUM_WORKERS = NUM_CORES * NUM_SUBCORES

CHUNK = 128
NBUF = 5
LOOKAHEAD = 3


def _make_gather(total_rows: int):
    assert total_rows % (NUM_WORKERS * CHUNK * NBUF) == 0
    rows_per_worker = total_rows // NUM_WORKERS
    n_chunks = rows_per_worker // CHUNK
    n_steps = n_chunks // NBUF
    assert n_steps >= 2

    mesh = plsc.VectorSubcoreMesh(core_axis_name="c", subcore_axis_name="s")

    @functools.partial(
        pl.kernel,
        mesh=mesh,
        out_type=jax.ShapeDtypeStruct((total_rows, HIDDEN), jnp.float32),
        scratch_types=[
            pltpu.VMEM((rows_per_worker,), jnp.int32),
        ]
        + [pltpu.VMEM((CHUNK, HIDDEN), jnp.float32) for _ in range(NBUF)]
        + [pltpu.SemaphoreType.DMA for _ in range(2 * NBUF)],
    )
    def gather_kernel(table_hbm, idx_hbm, out_hbm, idx_all, *bufs_and_sems):
        rows = bufs_and_sems[:NBUF]
        gsem = bufs_and_sems[NBUF:2 * NBUF]
        wsem = bufs_and_sems[2 * NBUF:]
        wid = lax.axis_index("s") * NUM_CORES + lax.axis_index("c")
        base0 = wid * rows_per_worker

        pltpu.sync_copy(idx_hbm.at[pl.ds(base0, rows_per_worker)], idx_all)

        def wait_writeback(b):
            pltpu.make_async_copy(
                rows[b], out_hbm.at[pl.ds(0, CHUNK)], wsem[b]
            ).wait()

        def issue_gather(s, b):
            pltpu.async_copy(
                table_hbm.at[idx_all.at[pl.ds(s * CHUNK, CHUNK)]],
                rows[b],
                gsem[b],
            )

        def drain_and_writeback(s, b):
            pltpu.make_async_copy(
                table_hbm.at[pl.ds(0, CHUNK)], rows[b], gsem[b]
            ).wait()
            pltpu.async_copy(
                rows[b], out_hbm.at[pl.ds(base0 + s * CHUNK, CHUNK)], wsem[b]
            )

        for s in range(LOOKAHEAD):
            issue_gather(s, s)

        for b in range(NBUF):
            drain_and_writeback(b, b)
            if b + LOOKAHEAD >= NBUF:
                wait_writeback((b + LOOKAHEAD) % NBUF)
            issue_gather(b + LOOKAHEAD, (b + LOOKAHEAD) % NBUF)

        def body(t, carry):
            for b in range(NBUF):
                s = NBUF * t + b
                drain_and_writeback(s, b)
                wait_writeback((b + LOOKAHEAD) % NBUF)
                issue_gather(s + LOOKAHEAD, (b + LOOKAHEAD) % NBUF)
            return carry

        lax.fori_loop(1, n_steps - 1, body, 0)

        for b in range(NBUF):
            s = NBUF * (n_steps - 1) + b
            drain_and_writeback(s, b)
            if b + LOOKAHEAD < NBUF:
                wait_writeback((b + LOOKAHEAD) % NBUF)
                issue_gather(s + LOOKAHEAD, (b + LOOKAHEAD) % NBUF)
        for b in range(NBUF):
            wait_writeback(b)

    return gather_kernel


def kernel(input_ids, embedding_weight):
    batch, seq = input_ids.shape
    ids = input_ids.reshape(-1).astype(jnp.int32)
    out = _make_gather(batch * seq)(embedding_weight, ids)
    return out.reshape(batch, seq, HIDDEN)
